# Initial kernel scaffold; baseline (speedup 1.0000x reference)
import jax
import jax.numpy as jnp
from jax.experimental import pallas as pl


def kernel(x, table, fc_w, fc_b):
    raise NotImplementedError("write your pallas kernel here")


# speedup vs baseline: 3.2817x; 3.2817x over previous
import functools

import jax
import jax.numpy as jnp
from jax import lax
from jax.experimental import pallas as pl
from jax.experimental.pallas import tpu as pltpu
from jax.experimental.pallas import tpu_sc as plsc

VOCAB = 1000000
EMBED = 64
B = 16384
L = 50

RB = 8000
NBLK = VOCAB // RB
RSUB = RB // 8

NW = 32
BPW = B // NW
NCHUNK = BPW // 16
IPW = BPW * L


def _v_kernel(w_ref, b_ref, t_ref, o_ref):
    w = w_ref[...]
    scale = 1.0 / L
    for s in range(8):
        t = t_ref[pl.ds(s * RSUB, RSUB), :]
        acc = lax.dot_general(w, t, (((1,), (1,)), ((), ())),
                              preferred_element_type=jnp.float32)
        o_ref[pl.ds(s, 1), :] = acc * scale + b_ref[0] * scale


def _compute_v(table, fc_w, fc_b):
    v2d = pl.pallas_call(
        _v_kernel,
        grid=(NBLK,),
        in_specs=[
            pl.BlockSpec((1, EMBED), lambda i: (0, 0)),
            pl.BlockSpec(memory_space=pltpu.SMEM),
            pl.BlockSpec((RB, EMBED), lambda i: (i, 0)),
        ],
        out_specs=pl.BlockSpec((8, RSUB), lambda i: (i, 0)),
        out_shape=jax.ShapeDtypeStruct((NBLK * 8, RSUB), jnp.float32),
    )(fc_w, fc_b, table)
    return v2d.reshape(VOCAB)


def _gather_kernel(v_hbm, idx_hbm, out_hbm, idx_v, vals_v, out_v, sem):
    nc = 2
    wid = lax.axis_index("s") * nc + lax.axis_index("c")
    pltpu.sync_copy(idx_hbm.at[wid], idx_v)
    pltpu.async_copy(v_hbm.at[idx_v], vals_v, sem).wait()
    for c in range(NCHUNK):
        base = c * 16 * L

        def body(l, acc, base=base):
            return acc + vals_v[pl.ds(base + l * 16, 16)]

        out_v[pl.ds(c * 16, 16)] = lax.fori_loop(
            0, L, body, jnp.zeros((16,), jnp.float32))
    pltpu.sync_copy(out_v, out_hbm.at[pl.ds(wid * BPW, BPW)])


_gather_call = functools.partial(
    pl.kernel,
    mesh=plsc.VectorSubcoreMesh(core_axis_name="c", subcore_axis_name="s"),
    out_type=jax.ShapeDtypeStruct((B,), jnp.float32),
    scratch_types=[
        pltpu.VMEM((IPW,), jnp.int32),
        pltpu.VMEM((IPW,), jnp.float32),
        pltpu.VMEM((BPW,), jnp.float32),
        pltpu.SemaphoreType.DMA,
    ],
)(_gather_kernel)


def kernel(x, table, fc_w, fc_b):
    x = x.astype(jnp.int32)
    idx = (x.reshape(NW, NCHUNK, 16, L)
             .transpose(0, 1, 3, 2)
             .reshape(NW, IPW))
    v = _compute_v(table, fc_w, fc_b)
    return _gather_call(v, idx)


# speedup vs baseline: 11.3070x; 3.4455x over previous
import functools

import jax
import jax.numpy as jnp
from jax import lax
from jax.experimental import pallas as pl
from jax.experimental.pallas import tpu as pltpu
from jax.experimental.pallas import tpu_sc as plsc

VOCAB = 1000000
EMBED = 64
B = 16384
L = 50

CB = 8192
NBLK = pl.cdiv(VOCAB, CB)
VP = NBLK * CB

NW = 32
BPW = B // NW
NCHUNK = BPW // 16


def _v_kernel(w_ref, b_ref, t_ref, o_ref):
    scale = 1.0 / L
    acc = lax.dot_general(w_ref[...], t_ref[...], (((1,), (0,)), ((), ())),
                          preferred_element_type=jnp.float32)
    i = pl.program_id(0)
    o_ref[pl.ds(i * CB, CB)] = jnp.reshape(acc * scale + b_ref[0] * scale,
                                           (CB,))


def _compute_v(table_t, fc_w, fc_b):
    return pl.pallas_call(
        _v_kernel,
        grid=(NBLK,),
        in_specs=[
            pl.BlockSpec((1, EMBED), lambda i: (0, 0)),
            pl.BlockSpec(memory_space=pltpu.SMEM),
            pl.BlockSpec((EMBED, CB), lambda i: (0, i)),
        ],
        out_specs=pl.BlockSpec((VP,), lambda i: (0,)),
        out_shape=jax.ShapeDtypeStruct((VP,), jnp.float32),
    )(fc_w, fc_b, table_t)


def _gather_kernel(v_hbm, xt_hbm, out_hbm, idx_v, vals_v, out_v, sem):
    nc = 2
    wid = lax.axis_index("s") * nc + lax.axis_index("c")
    copies = [pltpu.async_copy(xt_hbm.at[l, pl.ds(wid * BPW, BPW)],
                               idx_v.at[pl.ds(l * BPW, BPW)], sem)
              for l in range(L)]
    for c in copies:
        c.wait()
    pltpu.async_copy(v_hbm.at[idx_v], vals_v, sem).wait()
    for c in range(NCHUNK):
        def body(l, acc, c=c):
            return acc + vals_v[pl.ds(l * BPW + c * 16, 16)]

        out_v[pl.ds(c * 16, 16)] = lax.fori_loop(
            0, L, body, jnp.zeros((16,), jnp.float32))
    pltpu.sync_copy(out_v, out_hbm.at[pl.ds(wid * BPW, BPW)])


_gather_call = functools.partial(
    pl.kernel,
    mesh=plsc.VectorSubcoreMesh(core_axis_name="c", subcore_axis_name="s"),
    out_type=jax.ShapeDtypeStruct((B,), jnp.float32),
    scratch_types=[
        pltpu.VMEM((L * BPW,), jnp.int32),
        pltpu.VMEM((L * BPW,), jnp.float32),
        pltpu.VMEM((BPW,), jnp.float32),
        pltpu.SemaphoreType.DMA,
    ],
)(_gather_kernel)


def kernel(x, table, fc_w, fc_b):
    x = x.astype(jnp.int32)
    v = _compute_v(table.T, fc_w, fc_b)
    return _gather_call(v, x.T)


# speedup vs baseline: 14.7985x; 1.3088x over previous
import functools

import jax
import jax.numpy as jnp
from jax import lax
from jax.experimental import pallas as pl
from jax.experimental.pallas import tpu as pltpu
from jax.experimental.pallas import tpu_sc as plsc

VOCAB = 1000000
EMBED = 64
B = 16384
L = 50

CB = 65536
NBLK = pl.cdiv(VOCAB, CB)
VP = NBLK * CB

NW = 32
BPW = B // NW
NCHUNK = BPW // 16


def _v_kernel(w_ref, b_ref, t_ref, o_ref):
    scale = 1.0 / L
    acc = lax.dot_general(w_ref[...], t_ref[...], (((1,), (0,)), ((), ())),
                          preferred_element_type=jnp.float32)
    i = pl.program_id(0)
    o_ref[pl.ds(i * CB, CB)] = jnp.reshape(acc * scale + b_ref[0] * scale,
                                           (CB,))


def _compute_v(table_t, fc_w, fc_b):
    return pl.pallas_call(
        _v_kernel,
        grid=(NBLK,),
        in_specs=[
            pl.BlockSpec((1, EMBED), lambda i: (0, 0)),
            pl.BlockSpec(memory_space=pltpu.SMEM),
            pl.BlockSpec((EMBED, CB), lambda i: (0, i)),
        ],
        out_specs=pl.BlockSpec((VP,), lambda i: (0,)),
        out_shape=jax.ShapeDtypeStruct((VP,), jnp.float32),
    )(fc_w, fc_b, table_t)


def _gather_kernel(v_hbm, xt_hbm, out_hbm, idx_v, vals_v, out_v, sem):
    nc = 2
    wid = lax.axis_index("s") * nc + lax.axis_index("c")
    copies = [pltpu.async_copy(xt_hbm.at[l, pl.ds(wid * BPW, BPW)],
                               idx_v.at[pl.ds(l * BPW, BPW)], sem)
              for l in range(L)]
    for c in copies:
        c.wait()
    pltpu.async_copy(v_hbm.at[idx_v], vals_v, sem).wait()
    for c in range(NCHUNK):
        def body(l, acc, c=c):
            return acc + vals_v[pl.ds(l * BPW + c * 16, 16)]

        out_v[pl.ds(c * 16, 16)] = lax.fori_loop(
            0, L, body, jnp.zeros((16,), jnp.float32))
    pltpu.sync_copy(out_v, out_hbm.at[pl.ds(wid * BPW, BPW)])


_gather_call = functools.partial(
    pl.kernel,
    mesh=plsc.VectorSubcoreMesh(core_axis_name="c", subcore_axis_name="s"),
    out_type=jax.ShapeDtypeStruct((B,), jnp.float32),
    scratch_types=[
        pltpu.VMEM((L * BPW,), jnp.int32),
        pltpu.VMEM((L * BPW,), jnp.float32),
        pltpu.VMEM((BPW,), jnp.float32),
        pltpu.SemaphoreType.DMA,
    ],
)(_gather_kernel)


def kernel(x, table, fc_w, fc_b):
    x = x.astype(jnp.int32)
    v = _compute_v(table.T, fc_w, fc_b)
    return _gather_call(v, x.T)


# speedup vs baseline: 15.2442x; 1.0301x over previous
import functools

import jax
import jax.numpy as jnp
from jax import lax
from jax.experimental import pallas as pl
from jax.experimental.pallas import tpu as pltpu
from jax.experimental.pallas import tpu_sc as plsc

VOCAB = 1000000
EMBED = 64
B = 16384
L = 50

CB = 65536
NBLK = pl.cdiv(VOCAB, CB)
VP = NBLK * CB

NW = 32
BPW = B // NW
NCHUNK = BPW // 16


def _v_kernel(w_ref, b_ref, t_ref, o_ref):
    scale = 1.0 / L
    acc = lax.dot_general(w_ref[...], t_ref[...], (((1,), (0,)), ((), ())),
                          preferred_element_type=jnp.float32)
    i = pl.program_id(0)
    o_ref[pl.ds(i * CB, CB)] = jnp.reshape(acc * scale + b_ref[0] * scale,
                                           (CB,))


def _compute_v(table_t, fc_w, fc_b):
    return pl.pallas_call(
        _v_kernel,
        grid=(NBLK,),
        in_specs=[
            pl.BlockSpec((1, EMBED), lambda i: (0, 0)),
            pl.BlockSpec(memory_space=pltpu.SMEM),
            pl.BlockSpec((EMBED, CB), lambda i: (0, i)),
        ],
        out_specs=pl.BlockSpec((VP,), lambda i: (0,)),
        out_shape=jax.ShapeDtypeStruct((VP,), jnp.float32),
    )(fc_w, fc_b, table_t)


LCH = 10
NGC = L // LCH


def _gather_kernel(v_hbm, xt_hbm, out_hbm, i0, i1, i2, i3, i4,
                   vals_v, out_v, sem, gsem):
    nc = 2
    wid = lax.axis_index("s") * nc + lax.axis_index("c")
    idx_bufs = (i0, i1, i2, i3, i4)
    stage = [[pltpu.async_copy(
        xt_hbm.at[k * LCH + l, pl.ds(wid * BPW, BPW)],
        idx_bufs[k].at[pl.ds(l * BPW, BPW)], sem)
        for l in range(LCH)] for k in range(NGC)]
    gaths = []
    for k in range(NGC):
        for cp in stage[k]:
            cp.wait()
        gaths.append(pltpu.async_copy(
            v_hbm.at[idx_bufs[k]],
            vals_v.at[pl.ds(k * LCH * BPW, LCH * BPW)], gsem))
    for k in range(NGC):
        gaths[k].wait()
        for c in range(NCHUNK):
            def body(l, acc, k=k, c=c):
                return acc + vals_v[pl.ds((k * LCH + l) * BPW + c * 16, 16)]

            acc = lax.fori_loop(0, LCH, body, jnp.zeros((16,), jnp.float32))
            if k == 0:
                out_v[pl.ds(c * 16, 16)] = acc
            else:
                out_v[pl.ds(c * 16, 16)] = out_v[pl.ds(c * 16, 16)] + acc
    pltpu.sync_copy(out_v, out_hbm.at[pl.ds(wid * BPW, BPW)])


_gather_call = functools.partial(
    pl.kernel,
    mesh=plsc.VectorSubcoreMesh(core_axis_name="c", subcore_axis_name="s"),
    out_type=jax.ShapeDtypeStruct((B,), jnp.float32),
    scratch_types=[
        pltpu.VMEM((LCH * BPW,), jnp.int32),
        pltpu.VMEM((LCH * BPW,), jnp.int32),
        pltpu.VMEM((LCH * BPW,), jnp.int32),
        pltpu.VMEM((LCH * BPW,), jnp.int32),
        pltpu.VMEM((LCH * BPW,), jnp.int32),
        pltpu.VMEM((L * BPW,), jnp.float32),
        pltpu.VMEM((BPW,), jnp.float32),
        pltpu.SemaphoreType.DMA,
        pltpu.SemaphoreType.DMA,
    ],
)(_gather_kernel)


def kernel(x, table, fc_w, fc_b):
    x = x.astype(jnp.int32)
    v = _compute_v(table.T, fc_w, fc_b)
    return _gather_call(v, x.T)
